# baseline (speedup 1.0000x reference)
from functools import partial

import jax
import jax.numpy as jnp
from jax.experimental import pallas as pl
from jax.experimental.pallas import tpu as pltpu

_TOPK = 5
_KT = 2048


def _extract5(arrs):
    inf = jnp.float32(jnp.inf)
    vals, cnts = [], []
    for _ in range(_TOPK):
        m = None
        for a in arrs:
            am = jnp.min(a, axis=1, keepdims=True)
            m = am if m is None else jnp.minimum(m, am)
        cnt = None
        eqs = []
        for a in arrs:
            eq = a == m
            eqs.append(eq)
            c = jnp.sum(eq, axis=1, keepdims=True, dtype=jnp.float32)
            cnt = c if cnt is None else cnt + c
        arrs = [jnp.where(eq, inf, a) for a, eq in zip(arrs, eqs)]
        vals.append(m)
        cnts.append(cnt)
    return vals, cnts


def _rebuild(vals, cnts):
    cum = []
    c = jnp.zeros_like(cnts[0])
    for i in range(_TOPK):
        c = c + cnts[i]
        cum.append(c)
    cols = []
    for j in range(_TOPK):
        jj = jnp.float32(j)
        v = vals[_TOPK - 1]
        for i in range(_TOPK - 2, -1, -1):
            v = jnp.where(cum[i] > jj, vals[i], v)
        cols.append(v)
    return cols


def _shifted_tile(q, k):
    sim = jax.lax.dot_general(
        q, k, (((1,), (1,)), ((), ())), preferred_element_type=jnp.float32
    )
    ksq = k * k
    ones = jnp.ones((8, k.shape[1]), jnp.float32)
    k2all = jax.lax.dot_general(
        ones, ksq, (((1,), (1,)), ((), ())), preferred_element_type=jnp.float32
    )
    return k2all[0:1, :] - 2.0 * sim


def _fold_kernel(q_ref, k_ref, mean_ref, flag_ref, a1_ref, a2_ref, a3_ref,
                 a4_ref, *, n_tiles):
    t = pl.program_id(0)
    q = q_ref[...]

    @pl.when(t == 0)
    def _init():
        inf_tile = jnp.full(a1_ref.shape, jnp.inf, jnp.float32)
        a1_ref[...] = inf_tile
        a2_ref[...] = inf_tile
        a3_ref[...] = inf_tile
        a4_ref[...] = inf_tile

    s = _shifted_tile(q, k_ref[...])

    a1 = a1_ref[...]
    a2 = a2_ref[...]
    a3 = a3_ref[...]
    a4 = a4_ref[...]
    for g in range(s.shape[1] // 128):
        v = s[:, g * 128 : (g + 1) * 128]
        lo = jnp.minimum(a1, v)
        hi = jnp.maximum(a1, v)
        a1 = lo
        lo = jnp.minimum(a2, hi)
        hi = jnp.maximum(a2, hi)
        a2 = lo
        lo = jnp.minimum(a3, hi)
        hi = jnp.maximum(a3, hi)
        a3 = lo
        a4 = jnp.minimum(a4, hi)
    a1_ref[...] = a1
    a2_ref[...] = a2
    a3_ref[...] = a3
    a4_ref[...] = a4

    @pl.when(t == n_tiles - 1)
    def _finish():
        q2 = jnp.sum(q * q, axis=1, keepdims=True)
        vals, cnts = _extract5([a1, a2, a3, a4])
        cols = _rebuild(vals, cnts)
        v5 = cols[_TOPK - 1]
        s_out = cols[0]
        for j in range(1, _TOPK):
            s_out = s_out + cols[j]
        mean_ref[...] = s_out * jnp.float32(1.0 / _TOPK) + q2
        bad = jnp.sum((a4 < v5).astype(jnp.float32))
        flag_ref[...] = jnp.full(flag_ref.shape, bad, jnp.float32)


def _exact_kernel(q_ref, k_ref, mean_ref, best_ref, *, n_tiles):
    t = pl.program_id(0)
    q = q_ref[...]

    @pl.when(t == 0)
    def _init():
        best_ref[...] = jnp.full(best_ref.shape, jnp.inf, jnp.float32)

    s = _shifted_tile(q, k_ref[...])
    best = best_ref[...]
    vals, cnts = _extract5([s, best])
    cols = _rebuild(vals, cnts)
    for j in range(_TOPK):
        best_ref[:, j : j + 1] = cols[j]

    @pl.when(t == n_tiles - 1)
    def _finish():
        q2 = jnp.sum(q * q, axis=1, keepdims=True)
        s_out = cols[0]
        for j in range(1, _TOPK):
            s_out = s_out + cols[j]
        mean_ref[...] = s_out * jnp.float32(1.0 / _TOPK) + q2


def _norm_kernel(x_ref, o_ref):
    x = x_ref[...]
    vmin = jnp.min(x, axis=1, keepdims=True)
    vmax = jnp.max(x, axis=1, keepdims=True)
    o_ref[...] = (x - vmin) / (vmax - vmin + jnp.float32(1e-6))


@jax.jit
def kernel(feat_map, keys):
    B, C, H, W = feat_map.shape
    q = jnp.transpose(feat_map, (0, 2, 3, 1)).reshape(-1, C)
    N = q.shape[0]
    M = keys.shape[0]

    n_tiles = (M + _KT - 1) // _KT
    m_pad = n_tiles * _KT
    if m_pad != M:
        keys = jnp.concatenate(
            [keys, jnp.full((m_pad - M, C), 1e4, dtype=keys.dtype)], axis=0
        )

    common = dict(
        grid=(n_tiles,),
        in_specs=[
            pl.BlockSpec((N, C), lambda t: (0, 0)),
            pl.BlockSpec((_KT, C), lambda t: (t, 0)),
        ],
        compiler_params=pltpu.CompilerParams(dimension_semantics=("arbitrary",)),
    )

    mean, flag = pl.pallas_call(
        partial(_fold_kernel, n_tiles=n_tiles),
        out_specs=[
            pl.BlockSpec((N, 1), lambda t: (0, 0)),
            pl.BlockSpec((8, 128), lambda t: (0, 0)),
        ],
        out_shape=[
            jax.ShapeDtypeStruct((N, 1), jnp.float32),
            jax.ShapeDtypeStruct((8, 128), jnp.float32),
        ],
        scratch_shapes=[pltpu.VMEM((N, 128), jnp.float32) for _ in range(4)],
        **common,
    )(q, keys)

    def _slow(_):
        return pl.pallas_call(
            partial(_exact_kernel, n_tiles=n_tiles),
            out_specs=pl.BlockSpec((N, 1), lambda t: (0, 0)),
            out_shape=jax.ShapeDtypeStruct((N, 1), jnp.float32),
            scratch_shapes=[pltpu.VMEM((N, 128), jnp.float32)],
            **common,
        )(q, keys)

    mean = jax.lax.cond(flag[0, 0] > 0.0, _slow, lambda m: m, mean)

    hw = H * W
    tm = mean.reshape(B, hw)
    lanes = ((hw + 127) // 128) * 128
    tmp = jnp.concatenate(
        [tm, jnp.broadcast_to(tm[:, :1], (B, lanes - hw))], axis=1
    )
    amap = pl.pallas_call(
        _norm_kernel,
        out_shape=jax.ShapeDtypeStruct((B, lanes), jnp.float32),
    )(tmp)
    return amap[:, :hw].reshape(B, H, W)


# speedup vs baseline: 1.1660x; 1.1660x over previous
from functools import partial

import jax
import jax.numpy as jnp
from jax.experimental import pallas as pl
from jax.experimental.pallas import tpu as pltpu

_TOPK = 5
_KT = 2048


def _extract5(arrs):
    inf = jnp.float32(jnp.inf)
    vals, cnts = [], []
    for _ in range(_TOPK):
        m = None
        for a in arrs:
            am = jnp.min(a, axis=1, keepdims=True)
            m = am if m is None else jnp.minimum(m, am)
        cnt = None
        eqs = []
        for a in arrs:
            eq = a == m
            eqs.append(eq)
            c = jnp.sum(eq, axis=1, keepdims=True, dtype=jnp.float32)
            cnt = c if cnt is None else cnt + c
        arrs = [jnp.where(eq, inf, a) for a, eq in zip(arrs, eqs)]
        vals.append(m)
        cnts.append(cnt)
    return vals, cnts


def _rebuild(vals, cnts):
    cum = []
    c = jnp.zeros_like(cnts[0])
    for i in range(_TOPK):
        c = c + cnts[i]
        cum.append(c)
    cols = []
    for j in range(_TOPK):
        jj = jnp.float32(j)
        v = vals[_TOPK - 1]
        for i in range(_TOPK - 2, -1, -1):
            v = jnp.where(cum[i] > jj, vals[i], v)
        cols.append(v)
    return cols


def _shifted_tile(q, k):
    sim = jax.lax.dot_general(
        q, k, (((1,), (1,)), ((), ())), preferred_element_type=jnp.float32
    )
    ksq = k * k
    ones = jnp.ones((8, k.shape[1]), jnp.float32)
    k2all = jax.lax.dot_general(
        ones, ksq, (((1,), (1,)), ((), ())), preferred_element_type=jnp.float32
    )
    return k2all[0:1, :] - 2.0 * sim


def _fold_kernel(q_ref, k_ref, mean_ref, flag_ref, a1_ref, a2_ref, a3_ref,
                 a4_ref, *, n_tiles):
    t = pl.program_id(0)
    q = q_ref[...]

    @pl.when(t == 0)
    def _init():
        inf_tile = jnp.full(a1_ref.shape, jnp.inf, jnp.float32)
        a1_ref[...] = inf_tile
        a2_ref[...] = inf_tile
        a3_ref[...] = inf_tile
        a4_ref[...] = inf_tile

    s = _shifted_tile(q, k_ref[...])

    def cmpex(x, y):
        return jnp.minimum(x, y), jnp.maximum(x, y)

    def sort4(w, x, y, z):
        a, b = cmpex(w, x)
        c, d = cmpex(y, z)
        a, c = cmpex(a, c)
        b, d = cmpex(b, d)
        b, c = cmpex(b, c)
        return (a, b, c, d)

    def merge4(A, B):
        l1 = jnp.minimum(A[0], B[3])
        l2 = jnp.minimum(A[1], B[2])
        l3 = jnp.minimum(A[2], B[1])
        l4 = jnp.minimum(A[3], B[0])
        a, c = cmpex(l1, l3)
        b, d = cmpex(l2, l4)
        a, b = cmpex(a, b)
        c, d = cmpex(c, d)
        return (a, b, c, d)

    slices = [s[:, g * 128 : (g + 1) * 128] for g in range(s.shape[1] // 128)]
    quads = [
        sort4(slices[i], slices[i + 1], slices[i + 2], slices[i + 3])
        for i in range(0, len(slices), 4)
    ]
    while len(quads) > 1:
        quads = [
            merge4(quads[i], quads[i + 1]) for i in range(0, len(quads), 2)
        ]
    carry = (a1_ref[...], a2_ref[...], a3_ref[...], a4_ref[...])
    a1, a2, a3, a4 = merge4(quads[0], carry)
    a1_ref[...] = a1
    a2_ref[...] = a2
    a3_ref[...] = a3
    a4_ref[...] = a4

    @pl.when(t == n_tiles - 1)
    def _finish():
        q2 = jnp.sum(q * q, axis=1, keepdims=True)
        vals, cnts = _extract5([a1, a2, a3, a4])
        cols = _rebuild(vals, cnts)
        v5 = cols[_TOPK - 1]
        s_out = cols[0]
        for j in range(1, _TOPK):
            s_out = s_out + cols[j]
        mean_ref[...] = s_out * jnp.float32(1.0 / _TOPK) + q2
        bad = jnp.sum((a4 < v5).astype(jnp.float32))
        flag_ref[...] = jnp.full(flag_ref.shape, bad, jnp.float32)


def _exact_kernel(q_ref, k_ref, mean_ref, best_ref, *, n_tiles):
    t = pl.program_id(0)
    q = q_ref[...]

    @pl.when(t == 0)
    def _init():
        best_ref[...] = jnp.full(best_ref.shape, jnp.inf, jnp.float32)

    s = _shifted_tile(q, k_ref[...])
    best = best_ref[...]
    vals, cnts = _extract5([s, best])
    cols = _rebuild(vals, cnts)
    for j in range(_TOPK):
        best_ref[:, j : j + 1] = cols[j]

    @pl.when(t == n_tiles - 1)
    def _finish():
        q2 = jnp.sum(q * q, axis=1, keepdims=True)
        s_out = cols[0]
        for j in range(1, _TOPK):
            s_out = s_out + cols[j]
        mean_ref[...] = s_out * jnp.float32(1.0 / _TOPK) + q2


def _norm_kernel(x_ref, o_ref):
    x = x_ref[...]
    vmin = jnp.min(x, axis=1, keepdims=True)
    vmax = jnp.max(x, axis=1, keepdims=True)
    o_ref[...] = (x - vmin) / (vmax - vmin + jnp.float32(1e-6))


@jax.jit
def kernel(feat_map, keys):
    B, C, H, W = feat_map.shape
    q = jnp.transpose(feat_map, (0, 2, 3, 1)).reshape(-1, C)
    N = q.shape[0]
    M = keys.shape[0]

    n_tiles = (M + _KT - 1) // _KT
    m_pad = n_tiles * _KT
    if m_pad != M:
        keys = jnp.concatenate(
            [keys, jnp.full((m_pad - M, C), 1e4, dtype=keys.dtype)], axis=0
        )

    common = dict(
        grid=(n_tiles,),
        in_specs=[
            pl.BlockSpec((N, C), lambda t: (0, 0)),
            pl.BlockSpec((_KT, C), lambda t: (t, 0)),
        ],
        compiler_params=pltpu.CompilerParams(dimension_semantics=("arbitrary",)),
    )

    mean, flag = pl.pallas_call(
        partial(_fold_kernel, n_tiles=n_tiles),
        out_specs=[
            pl.BlockSpec((N, 1), lambda t: (0, 0)),
            pl.BlockSpec((8, 128), lambda t: (0, 0)),
        ],
        out_shape=[
            jax.ShapeDtypeStruct((N, 1), jnp.float32),
            jax.ShapeDtypeStruct((8, 128), jnp.float32),
        ],
        scratch_shapes=[pltpu.VMEM((N, 128), jnp.float32) for _ in range(4)],
        **common,
    )(q, keys)

    def _slow(_):
        return pl.pallas_call(
            partial(_exact_kernel, n_tiles=n_tiles),
            out_specs=pl.BlockSpec((N, 1), lambda t: (0, 0)),
            out_shape=jax.ShapeDtypeStruct((N, 1), jnp.float32),
            scratch_shapes=[pltpu.VMEM((N, 128), jnp.float32)],
            **common,
        )(q, keys)

    mean = jax.lax.cond(flag[0, 0] > 0.0, _slow, lambda m: m, mean)

    hw = H * W
    tm = mean.reshape(B, hw)
    lanes = ((hw + 127) // 128) * 128
    tmp = jnp.concatenate(
        [tm, jnp.broadcast_to(tm[:, :1], (B, lanes - hw))], axis=1
    )
    amap = pl.pallas_call(
        _norm_kernel,
        out_shape=jax.ShapeDtypeStruct((B, lanes), jnp.float32),
    )(tmp)
    return amap[:, :hw].reshape(B, H, W)


# speedup vs baseline: 1.6986x; 1.4567x over previous
from functools import partial

import jax
import jax.numpy as jnp
from jax.experimental import pallas as pl
from jax.experimental.pallas import tpu as pltpu

_TOPK = 5
_KT = 2048
_PAD_U = -1e30


def _extract5_max(arrs):
    ninf = jnp.float32(-jnp.inf)
    vals, cnts = [], []
    for _ in range(_TOPK):
        m = None
        for a in arrs:
            am = jnp.max(a, axis=1, keepdims=True)
            m = am if m is None else jnp.maximum(m, am)
        cnt = None
        eqs = []
        for a in arrs:
            eq = a == m
            eqs.append(eq)
            c = jnp.sum(eq, axis=1, keepdims=True, dtype=jnp.float32)
            cnt = c if cnt is None else cnt + c
        arrs = [jnp.where(eq, ninf, a) for a, eq in zip(arrs, eqs)]
        vals.append(m)
        cnts.append(cnt)
    return vals, cnts


def _rebuild(vals, cnts):
    cum = []
    c = jnp.zeros_like(cnts[0])
    for i in range(_TOPK):
        c = c + cnts[i]
        cum.append(c)
    cols = []
    for j in range(_TOPK):
        jj = jnp.float32(j)
        v = vals[_TOPK - 1]
        for i in range(_TOPK - 2, -1, -1):
            v = jnp.where(cum[i] > jj, vals[i], v)
        cols.append(v)
    return cols


def _u_tile(q, k):
    return jax.lax.dot_general(
        q, k, (((1,), (1,)), ((), ())), preferred_element_type=jnp.float32
    )


def _mean_from_cols(q, cols):
    q2 = jnp.sum(q * q, axis=1, keepdims=True) - 1.0
    s = cols[0]
    for j in range(1, _TOPK):
        s = s + cols[j]
    return q2 - 2.0 * s * jnp.float32(1.0 / _TOPK)


def _fold_kernel(q_ref, k_ref, mean_ref, flag_ref, a1_ref, a2_ref, a3_ref,
                 a4_ref, *, n_tiles):
    t = pl.program_id(0)

    @pl.when(t == 0)
    def _init():
        ninf_tile = jnp.full(a1_ref.shape, -jnp.inf, jnp.float32)
        a1_ref[...] = ninf_tile
        a2_ref[...] = ninf_tile
        a3_ref[...] = ninf_tile
        a4_ref[...] = ninf_tile

    u = _u_tile(q_ref[...], k_ref[...])

    def cmpex(x, y):
        return jnp.maximum(x, y), jnp.minimum(x, y)

    def sort4(w, x, y, z):
        a, b = cmpex(w, x)
        c, d = cmpex(y, z)
        a, c = cmpex(a, c)
        b, d = cmpex(b, d)
        b, c = cmpex(b, c)
        return (a, b, c, d)

    def merge4(A, B):
        l1 = jnp.maximum(A[0], B[3])
        l2 = jnp.maximum(A[1], B[2])
        l3 = jnp.maximum(A[2], B[1])
        l4 = jnp.maximum(A[3], B[0])
        a, c = cmpex(l1, l3)
        b, d = cmpex(l2, l4)
        a, b = cmpex(a, b)
        c, d = cmpex(c, d)
        return (a, b, c, d)

    slices = [u[:, g * 128 : (g + 1) * 128] for g in range(u.shape[1] // 128)]
    quads = [
        sort4(slices[i], slices[i + 1], slices[i + 2], slices[i + 3])
        for i in range(0, len(slices), 4)
    ]
    while len(quads) > 1:
        quads = [
            merge4(quads[i], quads[i + 1]) for i in range(0, len(quads), 2)
        ]
    carry = (a1_ref[...], a2_ref[...], a3_ref[...], a4_ref[...])
    a1, a2, a3, a4 = merge4(quads[0], carry)
    a1_ref[...] = a1
    a2_ref[...] = a2
    a3_ref[...] = a3
    a4_ref[...] = a4

    @pl.when(t == n_tiles - 1)
    def _finish():
        vals, cnts = _extract5_max([a1, a2, a3, a4])
        cols = _rebuild(vals, cnts)
        v5 = cols[_TOPK - 1]
        mean_ref[...] = _mean_from_cols(q_ref[...], cols)
        bad = jnp.sum((a4 > v5).astype(jnp.float32))
        flag_ref[...] = jnp.full(flag_ref.shape, bad, jnp.float32)


def _exact_kernel(q_ref, k_ref, mean_ref, best_ref, *, n_tiles):
    t = pl.program_id(0)

    @pl.when(t == 0)
    def _init():
        best_ref[...] = jnp.full(best_ref.shape, -jnp.inf, jnp.float32)

    u = _u_tile(q_ref[...], k_ref[...])
    best = best_ref[...]
    vals, cnts = _extract5_max([u, best])
    cols = _rebuild(vals, cnts)
    for j in range(_TOPK):
        best_ref[:, j : j + 1] = cols[j]

    @pl.when(t == n_tiles - 1)
    def _finish():
        mean_ref[...] = _mean_from_cols(q_ref[...], cols)


def _norm_kernel(x_ref, o_ref):
    x = x_ref[...]
    vmin = jnp.min(x, axis=1, keepdims=True)
    vmax = jnp.max(x, axis=1, keepdims=True)
    o_ref[...] = (x - vmin) / (vmax - vmin + jnp.float32(1e-6))


@jax.jit
def kernel(feat_map, keys):
    B, C, H, W = feat_map.shape
    q = jnp.transpose(feat_map, (0, 2, 3, 1)).reshape(-1, C)
    N = q.shape[0]
    M = keys.shape[0]

    q_aug = jnp.concatenate([q, jnp.ones((N, 1), q.dtype)], axis=1)
    k2 = jnp.sum(keys * keys, axis=1, keepdims=True)
    k_aug = jnp.concatenate([keys, -0.5 * k2], axis=1)

    n_tiles = (M + _KT - 1) // _KT
    m_pad = n_tiles * _KT
    if m_pad != M:
        pad = jnp.zeros((m_pad - M, C + 1), k_aug.dtype).at[:, C].set(_PAD_U)
        k_aug = jnp.concatenate([k_aug, pad], axis=0)

    common = dict(
        grid=(n_tiles,),
        in_specs=[
            pl.BlockSpec((N, C + 1), lambda t: (0, 0)),
            pl.BlockSpec((_KT, C + 1), lambda t: (t, 0)),
        ],
        compiler_params=pltpu.CompilerParams(dimension_semantics=("arbitrary",)),
    )

    mean, flag = pl.pallas_call(
        partial(_fold_kernel, n_tiles=n_tiles),
        out_specs=[
            pl.BlockSpec((N, 1), lambda t: (0, 0)),
            pl.BlockSpec((8, 128), lambda t: (0, 0)),
        ],
        out_shape=[
            jax.ShapeDtypeStruct((N, 1), jnp.float32),
            jax.ShapeDtypeStruct((8, 128), jnp.float32),
        ],
        scratch_shapes=[pltpu.VMEM((N, 128), jnp.float32) for _ in range(4)],
        **common,
    )(q_aug, k_aug)

    def _slow(_):
        return pl.pallas_call(
            partial(_exact_kernel, n_tiles=n_tiles),
            out_specs=pl.BlockSpec((N, 1), lambda t: (0, 0)),
            out_shape=jax.ShapeDtypeStruct((N, 1), jnp.float32),
            scratch_shapes=[pltpu.VMEM((N, 128), jnp.float32)],
            **common,
        )(q_aug, k_aug)

    mean = jax.lax.cond(flag[0, 0] > 0.0, _slow, lambda m: m, mean)

    hw = H * W
    tm = mean.reshape(B, hw)
    lanes = ((hw + 127) // 128) * 128
    tmp = jnp.concatenate(
        [tm, jnp.broadcast_to(tm[:, :1], (B, lanes - hw))], axis=1
    )
    amap = pl.pallas_call(
        _norm_kernel,
        out_shape=jax.ShapeDtypeStruct((B, lanes), jnp.float32),
    )(tmp)
    return amap[:, :hw].reshape(B, H, W)
